# Initial kernel scaffold; baseline (speedup 1.0000x reference)
import jax
import jax.numpy as jnp
from jax.experimental import pallas as pl


def kernel(entities, relations, edge_index):
    raise NotImplementedError("write your pallas kernel here")


# speedup vs baseline: 13.5240x; 13.5240x over previous
import functools

import jax
import jax.numpy as jnp
from jax import lax
from jax.experimental import pallas as pl
from jax.experimental.pallas import tpu as pltpu
from jax.experimental.pallas import tpu_sc as plsc

N_NODES = 10000
N_EDGES = 320000
D_FEAT = 128
D_HALF = D_FEAT // 2

N_TILES = 16
NP = 10240
NPT = NP // N_TILES
EPT = N_EDGES // N_TILES
K = 80
NCHUNKS = EPT // K
NBUF = 4
NMAIN = (NCHUNKS // NBUF) * NBUF
NOUTER = NMAIN // NBUF

_mesh = plsc.VectorSubcoreMesh(core_axis_name="c", subcore_axis_name="s")


def _mainloop(eh, srcall, dstall, acc, cnt, rows, ones_k, semg, sems, semc):
    def gather(ci, b):
        return pltpu.async_copy(eh.at[srcall.at[ci]], rows[b], semg[b])

    def gather_wait(ci, b):
        pltpu.make_async_copy(eh.at[srcall.at[ci]], rows[b], semg[b]).wait()

    for j in range(NBUF):
        gather(j, j)

    def outer(o, carry):
        for j in range(NBUF):
            ci = o * NBUF + j
            gather_wait(ci, j)
            sd = pltpu.async_copy(rows[j], acc.at[dstall.at[ci]], sems[j],
                                  add=True)

            @pl.when(o > 0)
            def _():
                pltpu.make_async_copy(ones_k, cnt.at[dstall.at[ci]],
                                      semc[j]).wait()

            pltpu.async_copy(ones_k, cnt.at[dstall.at[ci]], semc[j], add=True)

            sd.wait()

            @pl.when(o < NOUTER - 1)
            def _():
                gather(ci + NBUF, j)

        return carry

    lax.fori_loop(0, NOUTER, outer, 0)

    for ci in range(NMAIN, NCHUNKS):
        b = ci % NBUF
        pltpu.async_copy(eh.at[srcall.at[ci]], rows[b], semg[b]).wait()
        pltpu.async_copy(rows[b], acc.at[dstall.at[ci]], sems[b],
                         add=True).wait()
        pltpu.async_copy(ones_k, cnt.at[dstall.at[ci]], semc[b],
                         add=True).wait()

    for j in range(NBUF):
        pltpu.make_async_copy(ones_k, cnt.at[dstall.at[0]], semc[j]).wait()


def _ep_round(nrows, rowoff, base, c, acc, outbuf, scalebuf, out):
    pltpu.sync_copy(acc.at[pl.ds(base + rowoff, nrows)],
                    outbuf.at[pl.ds(0, nrows)])

    def grp(g, carry):
        sc16 = scalebuf[pl.ds(rowoff + g * 16, 16)]
        for l in range(16):
            scv = sc16[l]
            n = g * 16 + l
            for q in range(D_HALF // 16):
                outbuf[n, pl.ds(q * 16, 16)] = (
                    outbuf[n, pl.ds(q * 16, 16)] * scv)
        return carry

    lax.fori_loop(0, nrows // 16, grp, 0)

    pltpu.sync_copy(outbuf.at[pl.ds(0, nrows)],
                    out.at[c, pl.ds(base + rowoff, nrows)])


def _epilogue(nrows, base, c, acc, cnt, outbuf, cntbuf, scalebuf, out):
    pltpu.sync_copy(cnt.at[pl.ds(base, NPT)], cntbuf)

    def scl(q, carry):
        v = cntbuf[pl.ds(q * 16, 16)]
        sc = jnp.where(v > 0.0, 1.0 / jnp.maximum(v, 1.0), 0.0)
        scalebuf[pl.ds(q * 16, 16)] = sc
        return carry

    lax.fori_loop(0, NPT // 16, scl, 0)

    _ep_round(min(nrows, NPT // 2), 0, base, c, acc, outbuf, scalebuf, out)
    if nrows > NPT // 2:
        _ep_round(nrows - NPT // 2, NPT // 2, base, c, acc, outbuf, scalebuf,
                  out)


@functools.partial(
    pl.kernel,
    out_type=jax.ShapeDtypeStruct((2, N_NODES, D_HALF), jnp.float32),
    mesh=_mesh,
    compiler_params=pltpu.CompilerParams(use_tc_tiling_on_sc=False),
    scratch_types=[
        pltpu.VMEM_SHARED((NP, D_HALF), jnp.float32),
        pltpu.VMEM_SHARED((NP,), jnp.float32),
        pltpu.VMEM((NCHUNKS, K), jnp.int32),
        pltpu.VMEM((NCHUNKS, K), jnp.int32),
        [pltpu.VMEM((K, D_HALF), jnp.float32) for _ in range(NBUF)],
        pltpu.VMEM((K,), jnp.float32),
        pltpu.VMEM((NPT // 2, D_HALF), jnp.float32),
        pltpu.VMEM((NPT,), jnp.float32),
        pltpu.VMEM((NPT,), jnp.float32),
        [pltpu.SemaphoreType.DMA for _ in range(NBUF)],
        [pltpu.SemaphoreType.DMA for _ in range(NBUF)],
        [pltpu.SemaphoreType.DMA for _ in range(NBUF)],
    ],
)
def _dgnn_sc(eh0, eh1, src3, dst3, out, acc, cnt, srcall, dstall, rows,
             ones_k, outbuf, cntbuf, scalebuf, semg, sems, semc):
    c = lax.axis_index("c")
    s = lax.axis_index("s")
    base = s * NPT

    zv = jnp.zeros((16,), jnp.float32)
    ov = jnp.ones((16,), jnp.float32)

    def zrow(n, carry):
        for q in range(D_HALF // 16):
            outbuf[n, pl.ds(q * 16, 16)] = zv
        return carry

    lax.fori_loop(0, NPT // 2, zrow, 0)

    def zs(i, carry):
        scalebuf[pl.ds(i * 16, 16)] = zv
        return carry

    lax.fori_loop(0, NPT // 16, zs, 0)

    for j in range(K // 16):
        ones_k[pl.ds(j * 16, 16)] = ov

    pltpu.sync_copy(src3.at[s], srcall)
    pltpu.sync_copy(dst3.at[s], dstall)
    pltpu.sync_copy(outbuf, acc.at[pl.ds(base, NPT // 2)])
    pltpu.sync_copy(outbuf, acc.at[pl.ds(base + NPT // 2, NPT // 2)])
    pltpu.sync_copy(scalebuf, cnt.at[pl.ds(base, NPT)])
    plsc.subcore_barrier()

    @pl.when(c == 0)
    def _():
        _mainloop(eh0, srcall, dstall, acc, cnt, rows, ones_k, semg, sems,
                  semc)

    @pl.when(c == 1)
    def _():
        _mainloop(eh1, srcall, dstall, acc, cnt, rows, ones_k, semg, sems,
                  semc)

    plsc.subcore_barrier()

    @pl.when(s < N_TILES - 1)
    def _():
        _epilogue(NPT, base, c, acc, cnt, outbuf, cntbuf, scalebuf, out)

    @pl.when(s == N_TILES - 1)
    def _():
        _epilogue(N_NODES - (N_TILES - 1) * NPT, base, c, acc, cnt, outbuf,
                  cntbuf, scalebuf, out)


def kernel(entities, relations, edge_index):
    del relations
    src3 = edge_index[0].reshape(N_TILES, NCHUNKS, K)
    dst3 = edge_index[2].reshape(N_TILES, NCHUNKS, K)
    eh0 = entities[:, :D_HALF]
    eh1 = entities[:, D_HALF:]
    out3 = _dgnn_sc(eh0, eh1, src3, dst3)
    return out3.transpose(1, 0, 2).reshape(N_NODES, D_FEAT)


# speedup vs baseline: 18.7767x; 1.3884x over previous
import functools

import jax
import jax.numpy as jnp
from jax import lax
from jax.experimental import pallas as pl
from jax.experimental.pallas import tpu as pltpu
from jax.experimental.pallas import tpu_sc as plsc

N_NODES = 10000
N_EDGES = 320000
D_FEAT = 128
D_HALF = D_FEAT // 2

N_TILES = 16
NP = 10240
NPT = NP // N_TILES
EPT = N_EDGES // N_TILES
K = 80
NCHUNKS = EPT // K
NBUF = 4
NMAIN = (NCHUNKS // NBUF) * NBUF
NOUTER = NMAIN // NBUF

_mesh = plsc.VectorSubcoreMesh(core_axis_name="c", subcore_axis_name="s")


def _mainloop(eh2, srcall, dstall, acc, cnt, rows, ones_k, semg, sems, semc):
    def gather(ci, b):
        return pltpu.async_copy(eh2.at[srcall.at[ci]], rows[b], semg[b])

    def gather_wait(ci, b):
        pltpu.make_async_copy(eh2.at[srcall.at[ci]], rows[b], semg[b]).wait()

    for j in range(NBUF):
        gather(j, j)

    def outer(o, carry):
        for j in range(NBUF):
            ci = o * NBUF + j
            gather_wait(ci, j)
            sd = pltpu.async_copy(rows[j], acc.at[dstall.at[ci]], sems[j],
                                  add=True)

            @pl.when(o > 0)
            def _():
                pltpu.make_async_copy(ones_k, cnt.at[dstall.at[ci]],
                                      semc[j]).wait()

            pltpu.async_copy(ones_k, cnt.at[dstall.at[ci]], semc[j], add=True)

            sd.wait()

            @pl.when(o < NOUTER - 1)
            def _():
                gather(ci + NBUF, j)

        return carry

    lax.fori_loop(0, NOUTER, outer, 0)

    for ci in range(NMAIN, NCHUNKS):
        b = ci % NBUF
        pltpu.async_copy(eh2.at[srcall.at[ci]], rows[b], semg[b]).wait()
        pltpu.async_copy(rows[b], acc.at[dstall.at[ci]], sems[b],
                         add=True).wait()
        pltpu.async_copy(ones_k, cnt.at[dstall.at[ci]], semc[b],
                         add=True).wait()

    for j in range(NBUF):
        pltpu.make_async_copy(ones_k, cnt.at[dstall.at[0]], semc[j]).wait()


def _ep_round(nrows, rowoff, base, coloff, acc, outbuf, scalebuf, out):
    pltpu.sync_copy(acc.at[pl.ds(base + rowoff, nrows)],
                    outbuf.at[pl.ds(0, nrows)])

    def grp(g, carry):
        sc16 = scalebuf[pl.ds(rowoff + g * 16, 16)]
        for l in range(16):
            scv = sc16[l]
            n = g * 16 + l
            for q in range(D_HALF // 16):
                outbuf[n, pl.ds(q * 16, 16)] = (
                    outbuf[n, pl.ds(q * 16, 16)] * scv)
        return carry

    lax.fori_loop(0, nrows // 16, grp, 0)

    pltpu.sync_copy(outbuf.at[pl.ds(0, nrows)],
                    out.at[pl.ds(base + rowoff, nrows), pl.ds(coloff, D_HALF)])


def _epilogue(nrows, base, coloff, acc, cnt, outbuf, cntbuf, scalebuf, out):
    pltpu.sync_copy(cnt.at[pl.ds(base, NPT)], cntbuf)

    def scl(q, carry):
        v = cntbuf[pl.ds(q * 16, 16)]
        sc = jnp.where(v > 0.0, 1.0 / jnp.maximum(v, 1.0), 0.0)
        scalebuf[pl.ds(q * 16, 16)] = sc
        return carry

    lax.fori_loop(0, NPT // 16, scl, 0)

    _ep_round(min(nrows, NPT // 2), 0, base, coloff, acc, outbuf, scalebuf,
              out)
    if nrows > NPT // 2:
        _ep_round(nrows - NPT // 2, NPT // 2, base, coloff, acc, outbuf,
                  scalebuf, out)


@functools.partial(
    pl.kernel,
    out_type=jax.ShapeDtypeStruct((N_NODES, D_FEAT), jnp.float32),
    mesh=_mesh,
    compiler_params=pltpu.CompilerParams(use_tc_tiling_on_sc=False),
    scratch_types=[
        pltpu.VMEM_SHARED((NP, D_HALF), jnp.float32),
        pltpu.VMEM_SHARED((NP,), jnp.float32),
        pltpu.VMEM((NCHUNKS, K), jnp.int32),
        pltpu.VMEM((NCHUNKS, K), jnp.int32),
        [pltpu.VMEM((K, D_HALF), jnp.float32) for _ in range(NBUF)],
        pltpu.VMEM((K,), jnp.float32),
        pltpu.VMEM((NPT // 2, D_HALF), jnp.float32),
        pltpu.VMEM((NPT,), jnp.float32),
        pltpu.VMEM((NPT,), jnp.float32),
        [pltpu.SemaphoreType.DMA for _ in range(NBUF)],
        [pltpu.SemaphoreType.DMA for _ in range(NBUF)],
        [pltpu.SemaphoreType.DMA for _ in range(NBUF)],
    ],
)
def _dgnn_sc(eh2, ei4, out, acc, cnt, srcall, dstall, rows,
             ones_k, outbuf, cntbuf, scalebuf, semg, sems, semc):
    c = lax.axis_index("c")
    s = lax.axis_index("s")
    base = s * NPT

    zv = jnp.zeros((16,), jnp.float32)
    ov = jnp.ones((16,), jnp.float32)

    def zrow(n, carry):
        for q in range(D_HALF // 16):
            outbuf[n, pl.ds(q * 16, 16)] = zv
        return carry

    lax.fori_loop(0, NPT // 2, zrow, 0)

    def zs(i, carry):
        scalebuf[pl.ds(i * 16, 16)] = zv
        return carry

    lax.fori_loop(0, NPT // 16, zs, 0)

    for j in range(K // 16):
        ones_k[pl.ds(j * 16, 16)] = ov

    pltpu.sync_copy(ei4.at[0, s], srcall)
    pltpu.sync_copy(ei4.at[2, s], dstall)
    pltpu.sync_copy(outbuf, acc.at[pl.ds(base, NPT // 2)])
    pltpu.sync_copy(outbuf, acc.at[pl.ds(base + NPT // 2, NPT // 2)])
    pltpu.sync_copy(scalebuf, cnt.at[pl.ds(base, NPT)])

    def xform(ci, carry):
        for j in range(K // 16):
            v = srcall[ci, pl.ds(j * 16, 16)]
            srcall[ci, pl.ds(j * 16, 16)] = v * 2 + c
        return carry

    lax.fori_loop(0, NCHUNKS, xform, 0)

    plsc.subcore_barrier()

    _mainloop(eh2, srcall, dstall, acc, cnt, rows, ones_k, semg, sems, semc)

    plsc.subcore_barrier()

    coloff = c * D_HALF

    @pl.when(s < N_TILES - 1)
    def _():
        _epilogue(NPT, base, coloff, acc, cnt, outbuf, cntbuf, scalebuf, out)

    @pl.when(s == N_TILES - 1)
    def _():
        _epilogue(N_NODES - (N_TILES - 1) * NPT, base, coloff, acc, cnt,
                  outbuf, cntbuf, scalebuf, out)


def kernel(entities, relations, edge_index):
    del relations
    eh2 = entities.reshape(2 * N_NODES, D_HALF)
    ei4 = edge_index.reshape(3, N_TILES, NCHUNKS, K)
    return _dgnn_sc(eh2, ei4)
